# baseline (speedup 1.0000x reference)
import jax
import jax.numpy as jnp
from jax import lax
from jax.experimental import pallas as pl
from jax.experimental.pallas import tpu as pltpu
from jax.experimental.pallas import tpu_sc as plsc

B = 16384
N_EMB = 26
VOCAB = 100000
EDIM = 16
N_DENSE = 13
N_COL = N_EMB + N_DENSE
OUT_D = N_EMB * EDIM + N_DENSE
PACK = 128 // EDIM
TAB_ROWS = N_EMB * VOCAB // PACK

NC = 2
NS = 16
NW = NC * NS
ROWS_W = B // NW
R = 128
N_CHUNK = ROWS_W // R


def _body(in_hbm, tab_hbm, out_hbm, in_v, idx_v, sub_v, rows_v, block_v, sem):
    wid = lax.axis_index("s") * NC + lax.axis_index("c")

    def chunk(cc, carry):
        rowbase = wid * ROWS_W + cc * R
        pltpu.sync_copy(in_hbm.at[pl.ds(rowbase * N_COL, R * N_COL)], in_v)

        def col(i, c1):
            def vec(j, c2):
                flat = lax.iota(jnp.int32, 16) * N_COL + (j * (16 * N_COL) + i)
                vals = plsc.load_gather(in_v, [flat])
                full = vals.astype(jnp.int32) + i * VOCAB
                off = pl.multiple_of(j * 16, 16)
                idx_v[pl.ds(off, 16)] = lax.shift_right_logical(full, 3)
                sub_v[pl.ds(off, 16)] = lax.bitwise_and(full, 7)
                return c2

            lax.fori_loop(0, R // 16, vec, 0)
            pltpu.async_copy(tab_hbm.at[idx_v], rows_v, sem).wait()

            def place(j, c2):
                off = pl.multiple_of(j * 16, 16)
                rows16 = lax.iota(jnp.int32, 16) + j * 16
                sub16 = sub_v[pl.ds(off, 16)] * EDIM
                for d in range(EDIM):
                    vals = plsc.load_gather(rows_v, [rows16, sub16 + d])
                    plsc.store_scatter(
                        block_v,
                        [rows16, jnp.full((16,), i * EDIM + d, jnp.int32)],
                        vals,
                    )
                return c2

            lax.fori_loop(0, R // 16, place, 0)
            return c1

        lax.fori_loop(0, N_EMB, col, 0)

        def vecd(r, c3):
            k = lax.iota(jnp.int32, 16)
            ksrc = jnp.minimum(k, N_DENSE - 1)
            vals = plsc.load_gather(in_v, [r * N_COL + N_EMB + ksrc])
            plsc.store_scatter(
                block_v,
                [jnp.full((16,), r, jnp.int32), N_EMB * EDIM + ksrc],
                vals,
                mask=k < N_DENSE,
            )
            return c3

        lax.fori_loop(0, R, vecd, 0)
        pltpu.sync_copy(block_v, out_hbm.at[pl.ds(rowbase, R), :])
        return carry

    lax.fori_loop(0, N_CHUNK, chunk, 0)


def kernel(inputs, tables):
    in_flat = inputs.reshape(-1)
    tab8 = tables.reshape(TAB_ROWS, 128)
    mesh = plsc.VectorSubcoreMesh(core_axis_name="c", subcore_axis_name="s")
    k = pl.kernel(
        _body,
        out_type=jax.ShapeDtypeStruct((B, OUT_D), jnp.float32),
        mesh=mesh,
        scratch_types=[
            pltpu.VMEM((R * N_COL,), jnp.float32),
            pltpu.VMEM((R,), jnp.int32),
            pltpu.VMEM((R,), jnp.int32),
            pltpu.VMEM((R, 128), jnp.float32),
            pltpu.VMEM((R, OUT_D), jnp.float32),
            pltpu.SemaphoreType.DMA,
        ],
        compiler_params=pltpu.CompilerParams(
            use_tc_tiling_on_sc=True, needs_layout_passes=False
        ),
    )
    return k(in_flat, tab8)


# speedup vs baseline: 2.5230x; 2.5230x over previous
import jax
import jax.numpy as jnp
from jax import lax
from jax.experimental import pallas as pl
from jax.experimental.pallas import tpu as pltpu
from jax.experimental.pallas import tpu_sc as plsc

B = 16384
N_EMB = 26
VOCAB = 100000
EDIM = 16
N_DENSE = 13
N_COL = N_EMB + N_DENSE
OUT_D = N_EMB * EDIM + N_DENSE

NC = 2
NS = 16
NW = NC * NS
ROWS_W = B // NW

SLAB = 2048
N_FULL = VOCAB // SLAB
TAIL = VOCAB - N_FULL * SLAB
TAIL_A = 1664
TAIL_B = TAIL - TAIL_A
ITEMS = N_EMB * (N_FULL + 1)
V_CUT = N_FULL * SLAB + TAIL_A
T_STRIDE = V_CUT // 8
MAIN_G = N_EMB * T_STRIDE
TAB_G = MAIN_G + 8 * N_EMB


def _repack(src_load, n16, vbase, buf_v):
    def wstep(w, c):
        lv0 = vbase + w * 16
        k = lax.iota(jnp.int32, 16)
        row16 = lax.shift_right_logical(lv0 + k, 3)
        colb = lax.mul(lax.bitwise_and(k, 7), 16)
        for d in range(EDIM):
            vals = src_load(d, w)
            plsc.store_scatter(buf_v, [row16, colb + d], vals)
        return c

    return wstep


def _repack_body(tab_hbm, out_hbm, slab_v, tail_v, last_v, buf_v, sem):
    wid = lax.axis_index("s") * NC + lax.axis_index("c")
    n_mine = (ITEMS - wid + NW - 1) // NW

    def item(kk, carry):
        it = wid + kk * NW
        i = it // (N_FULL + 1)
        s = it % (N_FULL + 1)

        @pl.when(s < N_FULL)
        def _full():
            v0 = pl.multiple_of(s * SLAB, 128)
            pltpu.sync_copy(tab_hbm.at[i].at[:, pl.ds(v0, SLAB)], slab_v)

            def load(d, w):
                return slab_v[d, pl.ds(pl.multiple_of(w * 16, 16), 16)]

            lax.fori_loop(0, SLAB // 16, _repack(load, SLAB // 16, 0, buf_v), 0)
            g0 = pl.multiple_of(i * T_STRIDE + s * (SLAB // 8), 8)
            pltpu.sync_copy(buf_v, out_hbm.at[pl.ds(g0, SLAB // 8), :])

        @pl.when(s == N_FULL)
        def _tail():
            v0 = pl.multiple_of(N_FULL * SLAB, 128)
            pltpu.sync_copy(
                tab_hbm.at[i].at[:, pl.ds(v0, TAIL_A)], tail_v
            )
            for d in range(EDIM):
                pltpu.sync_copy(
                    tab_hbm.at[i].at[d, pl.ds(v0 + TAIL_A, TAIL_B)],
                    last_v.at[d],
                )

            def load_a(d, w):
                return tail_v[d, pl.ds(pl.multiple_of(w * 16, 16), 16)]

            lax.fori_loop(
                0, TAIL_A // 16, _repack(load_a, TAIL_A // 16, 0, buf_v), 0
            )

            def load_b(d, w):
                return last_v[
                    d, pl.ds(pl.multiple_of((w - TAIL_A // 16) * 16, 16), 16)
                ]

            lax.fori_loop(
                TAIL_A // 16,
                TAIL // 16,
                _repack(load_b, TAIL_B // 16, 0, buf_v),
                0,
            )
            g0 = pl.multiple_of(i * T_STRIDE + N_FULL * (SLAB // 8), 8)
            pltpu.sync_copy(
                buf_v.at[pl.ds(0, TAIL_A // 8), :],
                out_hbm.at[pl.ds(g0, TAIL_A // 8), :],
            )
            ga = pl.multiple_of(MAIN_G + 8 * i, 8)
            pltpu.sync_copy(
                buf_v.at[pl.ds(TAIL_A // 8, 8), :],
                out_hbm.at[pl.ds(ga, 8), :],
            )

        return carry

    lax.fori_loop(0, n_mine, item, 0)


def _gather_body(in_hbm, tab_hbm, out_hbm, in_v, idx_v, rows_v, dense_v, sem):
    wid = lax.axis_index("s") * NC + lax.axis_index("c")
    base = wid * ROWS_W

    pltpu.sync_copy(in_hbm.at[pl.ds(base, ROWS_W)], in_v)

    def vec_dense(j, c2):
        rows = lax.iota(jnp.int32, 16) + j * 16
        for c in range(N_DENSE):
            vals = plsc.load_gather(
                in_v, [rows, jnp.full((16,), N_EMB + c, jnp.int32)]
            )
            plsc.store_scatter(
                dense_v, [rows, jnp.full((16,), c, jnp.int32)], vals
            )
        return c2

    lax.fori_loop(0, ROWS_W // 16, vec_dense, 0)
    pltpu.sync_copy(
        dense_v,
        out_hbm.at[pl.ds(base, ROWS_W), pl.ds(N_EMB * EDIM, N_DENSE)],
    )

    def col(i, carry):
        def vec(j, c2):
            rows = lax.iota(jnp.int32, 16) + j * 16
            cols = jnp.full((16,), i, jnp.int32)
            vals = plsc.load_gather(in_v, [rows, cols])
            v = vals.astype(jnp.int32)
            main = v + i * V_CUT
            apx = (MAIN_G * 8 + 64 * i - V_CUT) + v
            off = pl.multiple_of(j * 16, 16)
            idx_v[pl.ds(off, 16)] = jnp.where(v < V_CUT, main, apx)
            return c2

        lax.fori_loop(0, ROWS_W // 16, vec, 0)
        pltpu.async_copy(tab_hbm.at[idx_v], rows_v, sem).wait()
        pltpu.sync_copy(
            rows_v,
            out_hbm.at[pl.ds(base, ROWS_W), pl.ds(i * EDIM, EDIM)],
        )
        return carry

    lax.fori_loop(0, N_EMB, col, 0)


def kernel(inputs, tables):
    mesh = plsc.VectorSubcoreMesh(core_axis_name="c", subcore_axis_name="s")

    repack = pl.kernel(
        _repack_body,
        out_type=jax.ShapeDtypeStruct((TAB_G, 128), jnp.float32),
        mesh=mesh,
        scratch_types=[
            pltpu.VMEM((EDIM, SLAB), jnp.float32),
            pltpu.VMEM((EDIM, TAIL_A), jnp.float32),
            pltpu.VMEM((EDIM, TAIL_B), jnp.float32),
            pltpu.VMEM((SLAB // 8, 128), jnp.float32),
            pltpu.SemaphoreType.DMA,
        ],
        compiler_params=pltpu.CompilerParams(
            use_tc_tiling_on_sc=True, needs_layout_passes=False
        ),
    )
    tab_lin = repack(tables.transpose(0, 2, 1)).reshape(TAB_G * 8, EDIM)

    gather = pl.kernel(
        _gather_body,
        out_type=jax.ShapeDtypeStruct((B, OUT_D), jnp.float32),
        mesh=mesh,
        scratch_types=[
            pltpu.VMEM((ROWS_W, N_COL), jnp.float32),
            pltpu.VMEM((ROWS_W,), jnp.int32),
            pltpu.VMEM((ROWS_W, EDIM), jnp.float32),
            pltpu.VMEM((ROWS_W, N_DENSE), jnp.float32),
            pltpu.SemaphoreType.DMA,
        ],
        compiler_params=pltpu.CompilerParams(
            use_tc_tiling_on_sc=False, needs_layout_passes=False
        ),
    )
    return gather(inputs, tab_lin)


# speedup vs baseline: 4.7207x; 1.8710x over previous
import jax
import jax.numpy as jnp
from jax import lax
from jax.experimental import pallas as pl
from jax.experimental.pallas import tpu as pltpu
from jax.experimental.pallas import tpu_sc as plsc

B = 16384
N_EMB = 26
VOCAB = 100000
EDIM = 16
N_DENSE = 13
N_COL = N_EMB + N_DENSE
OUT_D = N_EMB * EDIM + N_DENSE

NC = 2
NS = 16
NW = NC * NS
ROWS_W = B // NW

SLAB = 1024
N_FULL = VOCAB // SLAB
TAIL = VOCAB - N_FULL * SLAB
TAIL_A = 640
TAIL_B = TAIL - TAIL_A
ITEMS = N_EMB * N_FULL
V_CUT = N_FULL * SLAB + TAIL_A
T_STRIDE = V_CUT // 8
MAIN_G = N_EMB * T_STRIDE
TAB_G = MAIN_G + 8 * N_EMB


def _transpose16(src_load, buf_v, w0, n16):
    def wstep(w, c):
        lv0 = (w - w0) * 16
        k = lax.iota(jnp.int32, 16)
        row16 = lax.shift_right_logical(lv0 + k, 3)
        colb = lax.mul(lax.bitwise_and(k, 7), 16)
        vals = [src_load(d, w) for d in range(EDIM)]
        for d in range(EDIM):
            plsc.store_scatter(buf_v, [row16, colb + d], vals[d])
        return c

    return wstep


def _repack_body(
    tab_hbm, out_hbm, slab0, slab1, buf0, buf1, tail_v, last_v,
    semr0, semr1, semw0, semw1,
):
    wid = lax.axis_index("s") * NC + lax.axis_index("c")
    slabs = (slab0, slab1)
    bufs = (buf0, buf1)
    semr = (semr0, semr1)
    semw = (semw0, semw1)
    n_mine = (ITEMS - wid + NW - 1) // NW

    def coords(kk):
        it = wid + kk * NW
        return it // N_FULL, it % N_FULL

    def read(kk, p):
        i, s = coords(kk)
        v0 = pl.multiple_of(s * SLAB, 128)
        pltpu.async_copy(tab_hbm.at[i].at[:, pl.ds(v0, SLAB)], slabs[p], semr[p])

    def write(kk, p):
        i, s = coords(kk)
        g0 = pl.multiple_of(i * T_STRIDE + s * (SLAB // 8), 8)
        pltpu.async_copy(bufs[p], out_hbm.at[pl.ds(g0, SLAB // 8), :], semw[p])

    def wait_r(p):
        pltpu.make_async_copy(
            tab_hbm.at[0].at[:, pl.ds(0, SLAB)], slabs[p], semr[p]
        ).wait()

    def wait_w(p):
        pltpu.make_async_copy(
            bufs[p], out_hbm.at[pl.ds(0, SLAB // 8), :], semw[p]
        ).wait()

    read(0, 0)

    def pair(pp, carry):
        for p in range(2):
            kk = pp * 2 + p

            @pl.when(kk < n_mine)
            def _():
                wait_r(p)

                @pl.when(kk + 1 < n_mine)
                def _():
                    read(kk + 1, 1 - p)

                @pl.when(kk >= 2)
                def _():
                    wait_w(p)

                def load(d, w):
                    return slabs[p][d, pl.ds(pl.multiple_of(w * 16, 16), 16)]

                lax.fori_loop(
                    0, SLAB // 16, _transpose16(load, bufs[p], 0, SLAB // 16), 0
                )
                write(kk, p)

        return carry

    lax.fori_loop(0, (n_mine + 1) // 2, pair, 0)
    for p in range(2):
        @pl.when(n_mine > p)
        def _():
            wait_w(p)

    @pl.when(wid < N_EMB)
    def _tail():
        i = wid
        v0 = pl.multiple_of(N_FULL * SLAB, 128)
        pltpu.sync_copy(tab_hbm.at[i].at[:, pl.ds(v0, TAIL_A)], tail_v)
        for d in range(EDIM):
            pltpu.sync_copy(
                tab_hbm.at[i].at[d, pl.ds(v0 + TAIL_A, TAIL_B)], last_v.at[d]
            )

        def load_a(d, w):
            return tail_v[d, pl.ds(pl.multiple_of(w * 16, 16), 16)]

        lax.fori_loop(
            0, TAIL_A // 16, _transpose16(load_a, buf0, 0, TAIL_A // 16), 0
        )

        def load_b(d, w):
            return last_v[
                d, pl.ds(pl.multiple_of((w - TAIL_A // 16) * 16, 16), 16)
            ]

        lax.fori_loop(
            TAIL_A // 16,
            TAIL // 16,
            _transpose16(load_b, buf0, 0, TAIL_B // 16),
            0,
        )
        g0 = pl.multiple_of(i * T_STRIDE + N_FULL * (SLAB // 8), 8)
        pltpu.sync_copy(
            buf0.at[pl.ds(0, TAIL_A // 8), :],
            out_hbm.at[pl.ds(g0, TAIL_A // 8), :],
        )
        ga = pl.multiple_of(MAIN_G + 8 * i, 8)
        pltpu.sync_copy(
            buf0.at[pl.ds(TAIL_A // 8, 8), :], out_hbm.at[pl.ds(ga, 8), :]
        )


def _gather_body(in_hbm, tab_hbm, out_hbm, in_v, idx_v, rows_v, dense_v, sem):
    wid = lax.axis_index("s") * NC + lax.axis_index("c")
    base = wid * ROWS_W

    pltpu.sync_copy(in_hbm.at[pl.ds(base, ROWS_W)], in_v)

    def vec_dense(j, c2):
        rows = lax.iota(jnp.int32, 16) + j * 16
        for c in range(N_DENSE):
            vals = plsc.load_gather(
                in_v, [rows, jnp.full((16,), N_EMB + c, jnp.int32)]
            )
            plsc.store_scatter(
                dense_v, [rows, jnp.full((16,), c, jnp.int32)], vals
            )
        return c2

    lax.fori_loop(0, ROWS_W // 16, vec_dense, 0)
    pltpu.sync_copy(
        dense_v,
        out_hbm.at[pl.ds(base, ROWS_W), pl.ds(N_EMB * EDIM, N_DENSE)],
    )

    def col(i, carry):
        def vec(j, c2):
            rows = lax.iota(jnp.int32, 16) + j * 16
            cols = jnp.full((16,), i, jnp.int32)
            vals = plsc.load_gather(in_v, [rows, cols])
            v = vals.astype(jnp.int32)
            main = v + i * V_CUT
            apx = (MAIN_G * 8 + 64 * i - V_CUT) + v
            off = pl.multiple_of(j * 16, 16)
            idx_v[pl.ds(off, 16)] = jnp.where(v < V_CUT, main, apx)
            return c2

        lax.fori_loop(0, ROWS_W // 16, vec, 0)
        pltpu.async_copy(tab_hbm.at[idx_v], rows_v, sem).wait()
        pltpu.sync_copy(
            rows_v,
            out_hbm.at[pl.ds(base, ROWS_W), pl.ds(i * EDIM, EDIM)],
        )
        return carry

    lax.fori_loop(0, N_EMB, col, 0)


def kernel(inputs, tables):
    mesh = plsc.VectorSubcoreMesh(core_axis_name="c", subcore_axis_name="s")

    repack = pl.kernel(
        _repack_body,
        out_type=jax.ShapeDtypeStruct((TAB_G, 128), jnp.float32),
        mesh=mesh,
        scratch_types=[
            pltpu.VMEM((EDIM, SLAB), jnp.float32),
            pltpu.VMEM((EDIM, SLAB), jnp.float32),
            pltpu.VMEM((SLAB // 8, 128), jnp.float32),
            pltpu.VMEM((SLAB // 8, 128), jnp.float32),
            pltpu.VMEM((EDIM, TAIL_A), jnp.float32),
            pltpu.VMEM((EDIM, TAIL_B), jnp.float32),
            pltpu.SemaphoreType.DMA,
            pltpu.SemaphoreType.DMA,
            pltpu.SemaphoreType.DMA,
            pltpu.SemaphoreType.DMA,
        ],
        compiler_params=pltpu.CompilerParams(
            use_tc_tiling_on_sc=True, needs_layout_passes=False
        ),
    )
    tab_lin = repack(tables.transpose(0, 2, 1)).reshape(TAB_G * 8, EDIM)

    gather = pl.kernel(
        _gather_body,
        out_type=jax.ShapeDtypeStruct((B, OUT_D), jnp.float32),
        mesh=mesh,
        scratch_types=[
            pltpu.VMEM((ROWS_W, N_COL), jnp.float32),
            pltpu.VMEM((ROWS_W,), jnp.int32),
            pltpu.VMEM((ROWS_W, EDIM), jnp.float32),
            pltpu.VMEM((ROWS_W, N_DENSE), jnp.float32),
            pltpu.SemaphoreType.DMA,
        ],
        compiler_params=pltpu.CompilerParams(
            use_tc_tiling_on_sc=False, needs_layout_passes=False
        ),
    )
    return gather(inputs, tab_lin)


# speedup vs baseline: 5.1352x; 1.0878x over previous
import jax
import jax.numpy as jnp
from jax import lax
from jax.experimental import pallas as pl
from jax.experimental.pallas import tpu as pltpu
from jax.experimental.pallas import tpu_sc as plsc

B = 16384
N_EMB = 26
VOCAB = 100000
EDIM = 16
N_DENSE = 13
N_COL = N_EMB + N_DENSE
OUT_D = N_EMB * EDIM + N_DENSE

NC = 2
NS = 16
NW = NC * NS
ROWS_W = B // NW

SLAB = 1024
N_FULL = VOCAB // SLAB
TAIL = VOCAB - N_FULL * SLAB
TAIL_A = 640
TAIL_B = TAIL - TAIL_A
ITEMS = N_EMB * N_FULL
V_CUT = N_FULL * SLAB + TAIL_A
T_STRIDE = V_CUT // 8
MAIN_G = N_EMB * T_STRIDE
TAB_G = MAIN_G + 8 * N_EMB


def _transpose16(src_load, buf_v, w0, n16):
    def wstep(w, c):
        lv0 = (w - w0) * 16
        k = lax.iota(jnp.int32, 16)
        row16 = lax.shift_right_logical(lv0 + k, 3)
        colb = lax.mul(lax.bitwise_and(k, 7), 16)
        vals = [src_load(d, w) for d in range(EDIM)]
        for d in range(EDIM):
            plsc.store_scatter(buf_v, [row16, colb + d], vals[d])
        return c

    return wstep


def _repack_body(
    tab_hbm, out_hbm, slab0, slab1, buf0, buf1, tail_v, last_v,
    semr0, semr1, semw0, semw1,
):
    wid = lax.axis_index("s") * NC + lax.axis_index("c")
    slabs = (slab0, slab1)
    bufs = (buf0, buf1)
    semr = (semr0, semr1)
    semw = (semw0, semw1)
    n_mine = (ITEMS - wid + NW - 1) // NW

    def coords(kk):
        it = wid + kk * NW
        return it // N_FULL, it % N_FULL

    def read(kk, p):
        i, s = coords(kk)
        v0 = pl.multiple_of(s * SLAB, 128)
        pltpu.async_copy(tab_hbm.at[i].at[:, pl.ds(v0, SLAB)], slabs[p], semr[p])

    def write(kk, p):
        i, s = coords(kk)
        g0 = pl.multiple_of(i * T_STRIDE + s * (SLAB // 8), 8)
        pltpu.async_copy(bufs[p], out_hbm.at[pl.ds(g0, SLAB // 8), :], semw[p])

    def wait_r(p):
        pltpu.make_async_copy(
            tab_hbm.at[0].at[:, pl.ds(0, SLAB)], slabs[p], semr[p]
        ).wait()

    def wait_w(p):
        pltpu.make_async_copy(
            bufs[p], out_hbm.at[pl.ds(0, SLAB // 8), :], semw[p]
        ).wait()

    read(0, 0)

    def pair(pp, carry):
        for p in range(2):
            kk = pp * 2 + p

            @pl.when(kk < n_mine)
            def _():
                wait_r(p)

                @pl.when(kk + 1 < n_mine)
                def _():
                    read(kk + 1, 1 - p)

                @pl.when(kk >= 2)
                def _():
                    wait_w(p)

                def load(d, w):
                    return slabs[p][d, pl.ds(pl.multiple_of(w * 16, 16), 16)]

                lax.fori_loop(
                    0, SLAB // 16, _transpose16(load, bufs[p], 0, SLAB // 16), 0
                )
                write(kk, p)

        return carry

    lax.fori_loop(0, (n_mine + 1) // 2, pair, 0)
    for p in range(2):
        @pl.when(n_mine > p)
        def _():
            wait_w(p)

    @pl.when(wid < N_EMB)
    def _tail():
        i = wid
        v0 = pl.multiple_of(N_FULL * SLAB, 128)
        pltpu.sync_copy(tab_hbm.at[i].at[:, pl.ds(v0, TAIL_A)], tail_v)
        for d in range(EDIM):
            pltpu.sync_copy(
                tab_hbm.at[i].at[d, pl.ds(v0 + TAIL_A, TAIL_B)], last_v.at[d]
            )

        def load_a(d, w):
            return tail_v[d, pl.ds(pl.multiple_of(w * 16, 16), 16)]

        lax.fori_loop(
            0, TAIL_A // 16, _transpose16(load_a, buf0, 0, TAIL_A // 16), 0
        )

        def load_b(d, w):
            return last_v[
                d, pl.ds(pl.multiple_of((w - TAIL_A // 16) * 16, 16), 16)
            ]

        lax.fori_loop(
            TAIL_A // 16,
            TAIL // 16,
            _transpose16(load_b, buf0, 0, TAIL_B // 16),
            0,
        )
        g0 = pl.multiple_of(i * T_STRIDE + N_FULL * (SLAB // 8), 8)
        pltpu.sync_copy(
            buf0.at[pl.ds(0, TAIL_A // 8), :],
            out_hbm.at[pl.ds(g0, TAIL_A // 8), :],
        )
        ga = pl.multiple_of(MAIN_G + 8 * i, 8)
        pltpu.sync_copy(
            buf0.at[pl.ds(TAIL_A // 8, 8), :], out_hbm.at[pl.ds(ga, 8), :]
        )


def _gather_body(
    in_hbm, tab_hbm, out_hbm, in_v, idx0, idx1, rows0, rows1, dense_v,
    semg0, semg1, semw0, semw1,
):
    wid = lax.axis_index("s") * NC + lax.axis_index("c")
    base = wid * ROWS_W
    idxs = (idx0, idx1)
    rows = (rows0, rows1)
    semg = (semg0, semg1)
    semw = (semw0, semw1)

    pltpu.sync_copy(in_hbm.at[pl.ds(base, ROWS_W)], in_v)

    def vec_dense(j, c2):
        r16 = lax.iota(jnp.int32, 16) + j * 16
        for c in range(N_DENSE):
            vals = plsc.load_gather(
                in_v, [r16, jnp.full((16,), N_EMB + c, jnp.int32)]
            )
            plsc.store_scatter(
                dense_v, [r16, jnp.full((16,), c, jnp.int32)], vals
            )
        return c2

    lax.fori_loop(0, ROWS_W // 16, vec_dense, 0)
    pltpu.sync_copy(
        dense_v,
        out_hbm.at[pl.ds(base, ROWS_W), pl.ds(N_EMB * EDIM, N_DENSE)],
    )

    def build_idx(i, p):
        def vec(j, c2):
            r16 = lax.iota(jnp.int32, 16) + j * 16
            cols = jnp.full((16,), i, jnp.int32)
            vals = plsc.load_gather(in_v, [r16, cols])
            v = vals.astype(jnp.int32)
            main = v + i * V_CUT
            apx = (MAIN_G * 8 + 64 * i - V_CUT) + v
            off = pl.multiple_of(j * 16, 16)
            idxs[p][pl.ds(off, 16)] = jnp.where(v < V_CUT, main, apx)
            return c2

        lax.fori_loop(0, ROWS_W // 16, vec, 0)

    def fire_gather(p):
        pltpu.async_copy(tab_hbm.at[idxs[p]], rows[p], semg[p])

    def wait_gather(p):
        pltpu.make_async_copy(
            tab_hbm.at[idxs[p]], rows[p], semg[p]
        ).wait()

    def fire_write(i, p):
        pltpu.async_copy(
            rows[p],
            out_hbm.at[pl.ds(base, ROWS_W), pl.ds(i * EDIM, EDIM)],
            semw[p],
        )

    def wait_write(p):
        pltpu.make_async_copy(
            rows[p],
            out_hbm.at[pl.ds(base, ROWS_W), pl.ds(0, EDIM)],
            semw[p],
        ).wait()

    def pair(pp, carry):
        for p in range(2):
            i = pp * 2 + p

            @pl.when(i < N_EMB)
            def _():
                build_idx(i, p)

                @pl.when(i >= 2)
                def _():
                    wait_write(p)

                fire_gather(p)

                @pl.when(i >= 1)
                def _():
                    wait_gather(1 - p)
                    fire_write(i - 1, 1 - p)

        return carry

    lax.fori_loop(0, (N_EMB + 1) // 2, pair, 0)
    lastp = (N_EMB - 1) % 2
    wait_gather(lastp)
    fire_write(N_EMB - 1, lastp)
    wait_write(lastp)
    wait_write(1 - lastp)


def kernel(inputs, tables):
    mesh = plsc.VectorSubcoreMesh(core_axis_name="c", subcore_axis_name="s")

    repack = pl.kernel(
        _repack_body,
        out_type=jax.ShapeDtypeStruct((TAB_G, 128), jnp.float32),
        mesh=mesh,
        scratch_types=[
            pltpu.VMEM((EDIM, SLAB), jnp.float32),
            pltpu.VMEM((EDIM, SLAB), jnp.float32),
            pltpu.VMEM((SLAB // 8, 128), jnp.float32),
            pltpu.VMEM((SLAB // 8, 128), jnp.float32),
            pltpu.VMEM((EDIM, TAIL_A), jnp.float32),
            pltpu.VMEM((EDIM, TAIL_B), jnp.float32),
            pltpu.SemaphoreType.DMA,
            pltpu.SemaphoreType.DMA,
            pltpu.SemaphoreType.DMA,
            pltpu.SemaphoreType.DMA,
        ],
        compiler_params=pltpu.CompilerParams(
            use_tc_tiling_on_sc=True, needs_layout_passes=False
        ),
    )
    tab_lin = repack(tables.transpose(0, 2, 1)).reshape(TAB_G * 8, EDIM)

    gather = pl.kernel(
        _gather_body,
        out_type=jax.ShapeDtypeStruct((B, OUT_D), jnp.float32),
        mesh=mesh,
        scratch_types=[
            pltpu.VMEM((ROWS_W, N_COL), jnp.float32),
            pltpu.VMEM((ROWS_W,), jnp.int32),
            pltpu.VMEM((ROWS_W,), jnp.int32),
            pltpu.VMEM((ROWS_W, EDIM), jnp.float32),
            pltpu.VMEM((ROWS_W, EDIM), jnp.float32),
            pltpu.VMEM((ROWS_W, N_DENSE), jnp.float32),
            pltpu.SemaphoreType.DMA,
            pltpu.SemaphoreType.DMA,
            pltpu.SemaphoreType.DMA,
            pltpu.SemaphoreType.DMA,
        ],
        compiler_params=pltpu.CompilerParams(
            use_tc_tiling_on_sc=False, needs_layout_passes=False
        ),
    )
    return gather(inputs, tab_lin)


# speedup vs baseline: 5.4204x; 1.0555x over previous
import jax
import jax.numpy as jnp
from jax import lax
from jax.experimental import pallas as pl
from jax.experimental.pallas import tpu as pltpu
from jax.experimental.pallas import tpu_sc as plsc

B = 16384
N_EMB = 26
VOCAB = 100000
EDIM = 16
N_DENSE = 13
N_COL = N_EMB + N_DENSE
OUT_D = N_EMB * EDIM + N_DENSE

NC = 2
NS = 16
NW = NC * NS
ROWS_W = B // NW

SLAB = 1536
N_FULL = VOCAB // SLAB
TAIL = VOCAB - N_FULL * SLAB
TAIL_A = 128
TAIL_B = TAIL - TAIL_A
ITEMS = N_EMB * N_FULL
V_CUT = N_FULL * SLAB + TAIL_A
T_STRIDE = V_CUT // 8
MAIN_G = N_EMB * T_STRIDE
TAB_G = MAIN_G + 8 * N_EMB


def _transpose16(src_load, buf_v, w0, n16):
    def wstep(w, c):
        lv0 = (w - w0) * 16
        k = lax.iota(jnp.int32, 16)
        row16 = lax.shift_right_logical(lv0 + k, 3)
        colb = lax.mul(lax.bitwise_and(k, 7), 16)
        vals = [src_load(d, w) for d in range(EDIM)]
        for d in range(EDIM):
            plsc.store_scatter(buf_v, [row16, colb + d], vals[d])
        return c

    return wstep


def _repack_body(
    tab_hbm, out_hbm, slab0, slab1, buf0, buf1, tail_v, last_v,
    semr0, semr1, semw0, semw1,
):
    wid = lax.axis_index("s") * NC + lax.axis_index("c")
    slabs = (slab0, slab1)
    bufs = (buf0, buf1)
    semr = (semr0, semr1)
    semw = (semw0, semw1)
    n_mine = (ITEMS - wid + NW - 1) // NW

    def coords(kk):
        it = wid + kk * NW
        return it // N_FULL, it % N_FULL

    def read(kk, p):
        i, s = coords(kk)
        v0 = pl.multiple_of(s * SLAB, 128)
        pltpu.async_copy(tab_hbm.at[i].at[:, pl.ds(v0, SLAB)], slabs[p], semr[p])

    def write(kk, p):
        i, s = coords(kk)
        g0 = pl.multiple_of(i * T_STRIDE + s * (SLAB // 8), 8)
        pltpu.async_copy(bufs[p], out_hbm.at[pl.ds(g0, SLAB // 8), :], semw[p])

    def wait_r(p):
        pltpu.make_async_copy(
            tab_hbm.at[0].at[:, pl.ds(0, SLAB)], slabs[p], semr[p]
        ).wait()

    def wait_w(p):
        pltpu.make_async_copy(
            bufs[p], out_hbm.at[pl.ds(0, SLAB // 8), :], semw[p]
        ).wait()

    read(0, 0)

    def pair(pp, carry):
        for p in range(2):
            kk = pp * 2 + p

            @pl.when(kk < n_mine)
            def _():
                wait_r(p)

                @pl.when(kk + 1 < n_mine)
                def _():
                    read(kk + 1, 1 - p)

                @pl.when(kk >= 2)
                def _():
                    wait_w(p)

                def load(d, w):
                    return slabs[p][d, pl.ds(pl.multiple_of(w * 16, 16), 16)]

                lax.fori_loop(
                    0, SLAB // 16, _transpose16(load, bufs[p], 0, SLAB // 16), 0
                )
                write(kk, p)

        return carry

    lax.fori_loop(0, (n_mine + 1) // 2, pair, 0)
    for p in range(2):
        @pl.when(n_mine > p)
        def _():
            wait_w(p)

    @pl.when(wid < N_EMB)
    def _tail():
        i = wid
        v0 = pl.multiple_of(N_FULL * SLAB, 128)
        pltpu.sync_copy(tab_hbm.at[i].at[:, pl.ds(v0, TAIL_A)], tail_v)
        for d in range(EDIM):
            pltpu.sync_copy(
                tab_hbm.at[i].at[d, pl.ds(v0 + TAIL_A, TAIL_B)], last_v.at[d]
            )

        def load_a(d, w):
            return tail_v[d, pl.ds(pl.multiple_of(w * 16, 16), 16)]

        lax.fori_loop(
            0, TAIL_A // 16, _transpose16(load_a, buf0, 0, TAIL_A // 16), 0
        )

        def load_b(d, w):
            return last_v[
                d, pl.ds(pl.multiple_of((w - TAIL_A // 16) * 16, 16), 16)
            ]

        lax.fori_loop(
            TAIL_A // 16,
            TAIL // 16,
            _transpose16(load_b, buf0, 0, TAIL_B // 16),
            0,
        )
        g0 = pl.multiple_of(i * T_STRIDE + N_FULL * (SLAB // 8), 8)
        pltpu.sync_copy(
            buf0.at[pl.ds(0, TAIL_A // 8), :],
            out_hbm.at[pl.ds(g0, TAIL_A // 8), :],
        )
        ga = pl.multiple_of(MAIN_G + 8 * i, 8)
        pltpu.sync_copy(
            buf0.at[pl.ds(TAIL_A // 8, 8), :], out_hbm.at[pl.ds(ga, 8), :]
        )


def _gather_body(
    in_hbm, tab_hbm, out_hbm, in_v, idx0, idx1, rows0, rows1, dense_v,
    semg0, semg1, semw0, semw1,
):
    wid = lax.axis_index("s") * NC + lax.axis_index("c")
    base = wid * ROWS_W
    idxs = (idx0, idx1)
    rows = (rows0, rows1)
    semg = (semg0, semg1)
    semw = (semw0, semw1)

    pltpu.sync_copy(in_hbm.at[pl.ds(base, ROWS_W)], in_v)

    def vec_dense(j, c2):
        r16 = lax.iota(jnp.int32, 16) + j * 16
        for c in range(N_DENSE):
            vals = plsc.load_gather(
                in_v, [r16, jnp.full((16,), N_EMB + c, jnp.int32)]
            )
            plsc.store_scatter(
                dense_v, [r16, jnp.full((16,), c, jnp.int32)], vals
            )
        return c2

    lax.fori_loop(0, ROWS_W // 16, vec_dense, 0)
    pltpu.sync_copy(
        dense_v,
        out_hbm.at[pl.ds(base, ROWS_W), pl.ds(N_EMB * EDIM, N_DENSE)],
    )

    def build_idx(i, p):
        def vec(j, c2):
            r16 = lax.iota(jnp.int32, 16) + j * 16
            cols = jnp.full((16,), i, jnp.int32)
            vals = plsc.load_gather(in_v, [r16, cols])
            v = vals.astype(jnp.int32)
            main = v + i * V_CUT
            apx = (MAIN_G * 8 + 64 * i - V_CUT) + v
            off = pl.multiple_of(j * 16, 16)
            idxs[p][pl.ds(off, 16)] = jnp.where(v < V_CUT, main, apx)
            return c2

        lax.fori_loop(0, ROWS_W // 16, vec, 0)

    def fire_gather(p):
        pltpu.async_copy(tab_hbm.at[idxs[p]], rows[p], semg[p])

    def wait_gather(p):
        pltpu.make_async_copy(
            tab_hbm.at[idxs[p]], rows[p], semg[p]
        ).wait()

    def fire_write(i, p):
        pltpu.async_copy(
            rows[p],
            out_hbm.at[pl.ds(base, ROWS_W), pl.ds(i * EDIM, EDIM)],
            semw[p],
        )

    def wait_write(p):
        pltpu.make_async_copy(
            rows[p],
            out_hbm.at[pl.ds(base, ROWS_W), pl.ds(0, EDIM)],
            semw[p],
        ).wait()

    def pair(pp, carry):
        for p in range(2):
            i = pp * 2 + p

            @pl.when(i < N_EMB)
            def _():
                build_idx(i, p)

                @pl.when(i >= 2)
                def _():
                    wait_write(p)

                fire_gather(p)

                @pl.when(i >= 1)
                def _():
                    wait_gather(1 - p)
                    fire_write(i - 1, 1 - p)

        return carry

    lax.fori_loop(0, (N_EMB + 1) // 2, pair, 0)
    lastp = (N_EMB - 1) % 2
    wait_gather(lastp)
    fire_write(N_EMB - 1, lastp)
    wait_write(lastp)
    wait_write(1 - lastp)


def kernel(inputs, tables):
    mesh = plsc.VectorSubcoreMesh(core_axis_name="c", subcore_axis_name="s")

    repack = pl.kernel(
        _repack_body,
        out_type=jax.ShapeDtypeStruct((TAB_G, 128), jnp.float32),
        mesh=mesh,
        scratch_types=[
            pltpu.VMEM((EDIM, SLAB), jnp.float32),
            pltpu.VMEM((EDIM, SLAB), jnp.float32),
            pltpu.VMEM((SLAB // 8, 128), jnp.float32),
            pltpu.VMEM((SLAB // 8, 128), jnp.float32),
            pltpu.VMEM((EDIM, TAIL_A), jnp.float32),
            pltpu.VMEM((EDIM, TAIL_B), jnp.float32),
            pltpu.SemaphoreType.DMA,
            pltpu.SemaphoreType.DMA,
            pltpu.SemaphoreType.DMA,
            pltpu.SemaphoreType.DMA,
        ],
        compiler_params=pltpu.CompilerParams(
            use_tc_tiling_on_sc=True, needs_layout_passes=False
        ),
    )
    tab_lin = repack(tables.transpose(0, 2, 1)).reshape(TAB_G * 8, EDIM)

    gather = pl.kernel(
        _gather_body,
        out_type=jax.ShapeDtypeStruct((B, OUT_D), jnp.float32),
        mesh=mesh,
        scratch_types=[
            pltpu.VMEM((ROWS_W, N_COL), jnp.float32),
            pltpu.VMEM((ROWS_W,), jnp.int32),
            pltpu.VMEM((ROWS_W,), jnp.int32),
            pltpu.VMEM((ROWS_W, EDIM), jnp.float32),
            pltpu.VMEM((ROWS_W, EDIM), jnp.float32),
            pltpu.VMEM((ROWS_W, N_DENSE), jnp.float32),
            pltpu.SemaphoreType.DMA,
            pltpu.SemaphoreType.DMA,
            pltpu.SemaphoreType.DMA,
            pltpu.SemaphoreType.DMA,
        ],
        compiler_params=pltpu.CompilerParams(
            use_tc_tiling_on_sc=False, needs_layout_passes=False
        ),
    )
    return gather(inputs, tab_lin)
